# baseline (speedup 1.0000x reference)
import functools

import jax
import jax.numpy as jnp
from jax import lax
from jax.experimental import pallas as pl
from jax.experimental.pallas import tpu as pltpu
from jax.experimental.pallas import tpu_sc as plsc

BINS0, BINS1 = 256, 256
CH = 4
ROWW = BINS1 * CH
INV_D = 128.0
NC, NS, L = 2, 16, 16
NW = NC * NS
HB = 56
_LOS = list(range(0, BINS0, HB))
_HS = [min(HB, BINS0 - lo) for lo in _LOS]
NPASS = len(_LOS)


def _sc_body(n_points, spw, bt_hbm, out_hbm,
             pts_v, grid_a, grid_b, sem_a, sem_b, sem_x):
    nvec = n_points // L
    wid = lax.axis_index("s") * NC + lax.axis_index("c")
    onesf = jnp.ones((L,), jnp.float32)
    zerof = jnp.zeros((L,), jnp.float32)
    lanes = lax.iota(jnp.int32, L)
    zeros_i = jnp.zeros((L,), jnp.int32)
    ones_i = jnp.ones((L,), jnp.int32)
    grids = [grid_a, grid_b]
    sems = [sem_a, sem_b]

    n2 = n_points * 2
    b0 = wid * spw
    cpp = pltpu.async_copy(bt_hbm.at[pl.ds(b0 * n2, spw * n2)], pts_v, sem_x)

    def zrow(r, _):
        for j in range(ROWW // L):
            grid_a[r, pl.ds(j * L, L)] = zerof
            grid_b[r, pl.ds(j * L, L)] = zerof
        return 0
    lax.fori_loop(0, HB, zrow, 0)

    cpp.wait()

    def pass_scatter(grid_v, s, lo, hi, forward):
        sbase = s * n_points * 2

        def it(i, _):
            eidx = sbase + (i * L + lanes) * 2
            xv = plsc.load_gather(pts_v, [eidx])
            yv = plsc.load_gather(pts_v, [eidx + 1])
            ixv = (xv * INV_D + 0.5).astype(jnp.int32)
            iyv = (yv * INV_D + 0.5).astype(jnp.int32)
            m = jnp.logical_and(ixv >= lo, ixv < hi)
            r = ixv - lo
            c = iyv * CH
            v2 = xv if forward else zerof
            v3 = yv if forward else zerof
            v01 = onesf if forward else zerof
            plsc.store_scatter(grid_v, [r, c], v01, mask=m)
            plsc.store_scatter(grid_v, [r, c + 1], v01, mask=m)
            plsc.store_scatter(grid_v, [r, c + 2], v2, mask=m)
            plsc.store_scatter(grid_v, [r, c + 3], v3, mask=m)
            return 0
        lax.fori_loop(0, nvec, it, 0, unroll=4)

    all_has = []
    for s in range(spw):
        sbase = s * n_points * 2

        def minmax(i, carry):
            mn, mx = carry
            xv = plsc.load_gather(pts_v, [sbase + (i * L + lanes) * 2])
            ix = (xv * INV_D + 0.5).astype(jnp.int32)
            return jnp.minimum(mn, ix), jnp.maximum(mx, ix)

        big = jnp.full((L,), 2**30, jnp.int32)
        mn_v, mx_v = lax.fori_loop(0, nvec, minmax, (big, -big), unroll=4)
        mn = jnp.min(mn_v)
        mx = jnp.max(mx_v)
        all_has.append([jnp.logical_and(mx >= lo, mn < lo + h)
                        for lo, h in zip(_LOS, _HS)])

    total = spw * NPASS
    for g in range(total):
        s, p = divmod(g, NPASS)
        k = g % 2
        lo, h = _LOS[p], _HS[p]
        if g >= 2:
            ps, pp = divmod(g - 2, NPASS)
            pltpu.make_async_copy(
                grids[k].at[pl.ds(0, _HS[pp])],
                out_hbm.at[b0 + ps, pl.ds(_LOS[pp], _HS[pp])],
                sems[k]).wait()

            @pl.when(all_has[ps][pp])
            def _cleanup():
                pass_scatter(grids[k], ps, _LOS[pp],
                             _LOS[pp] + _HS[pp], False)

        @pl.when(all_has[s][p])
        def _scatter():
            pass_scatter(grids[k], s, lo, lo + h, True)

        pltpu.async_copy(
            grids[k].at[pl.ds(0, h)],
            out_hbm.at[b0 + s, pl.ds(lo, h)],
            sems[k])

    for g in (total - 2, total - 1):
        s, p = divmod(g, NPASS)
        k = g % 2
        pltpu.make_async_copy(
            grids[k].at[pl.ds(0, _HS[p])],
            out_hbm.at[b0 + s, pl.ds(_LOS[p], _HS[p])],
            sems[k]).wait()


@functools.lru_cache(maxsize=None)
def _build(batch_size, n_points):
    spw = batch_size // NW
    mesh = plsc.VectorSubcoreMesh(
        core_axis_name="c", subcore_axis_name="s",
        num_cores=NC, num_subcores=NS)
    return pl.kernel(
        functools.partial(_sc_body, n_points, spw),
        out_type=jax.ShapeDtypeStruct((batch_size, BINS0, ROWW), jnp.float32),
        mesh=mesh,
        compiler_params=pltpu.CompilerParams(needs_layout_passes=False),
        scratch_types=[
            pltpu.VMEM((spw * n_points * 2,), jnp.float32),
            pltpu.VMEM((HB, ROWW), jnp.float32),
            pltpu.VMEM((HB, ROWW), jnp.float32),
            pltpu.SemaphoreType.DMA,
            pltpu.SemaphoreType.DMA,
            pltpu.SemaphoreType.DMA,
        ],
    )


def kernel(batch):
    batch_size, n_points, _ = batch.shape
    flat = batch.reshape(batch_size * n_points * 2)
    out = _build(batch_size, n_points)(flat)
    return out.reshape(batch_size, BINS0, BINS1, CH)


# speedup vs baseline: 4.4658x; 4.4658x over previous
import functools

import jax
import jax.numpy as jnp
from jax import lax
from jax.experimental import pallas as pl
from jax.experimental.pallas import tpu as pltpu
from jax.experimental.pallas import tpu_sc as plsc

BINS0, BINS1 = 256, 256
CH = 4
ROWW = BINS1 * CH
INV_D = 128.0
NC, NS, L = 2, 16, 16
NW = NC * NS
HB = 56
_LOS = list(range(0, BINS0, HB))
_HS = [min(HB, BINS0 - lo) for lo in _LOS]
NPASS = len(_LOS)


def _sc_body(n_points, spw, bt_hbm, out_hbm,
             pts_v, grid_a, grid_b, sem_a, sem_b, sem_x):
    nvec = n_points // L
    wid = lax.axis_index("s") * NC + lax.axis_index("c")
    onesf = jnp.ones((L,), jnp.float32)
    zerof = jnp.zeros((L,), jnp.float32)
    lanes = lax.iota(jnp.int32, L)
    zeros_i = jnp.zeros((L,), jnp.int32)
    ones_i = jnp.ones((L,), jnp.int32)
    grids = [grid_a, grid_b]
    sems = [sem_a, sem_b]

    n2 = n_points * 2
    b0 = wid * spw
    cpp = pltpu.async_copy(bt_hbm.at[pl.ds(b0 * n2, spw * n2)], pts_v, sem_x)

    def zrow(r, _):
        for q in range(8):
            for j in range(128 // L):
                grid_a[r, q, pl.ds(j * L, L)] = zerof
                grid_b[r, q, pl.ds(j * L, L)] = zerof
        return 0
    lax.fori_loop(0, HB, zrow, 0)

    cpp.wait()

    def pass_scatter(grid_v, s, lo, hi, forward):
        sbase = s * n_points * 2

        def it(i, _):
            base = sbase + (i // 8) * 256 + (i % 8) * L
            xv = pts_v[pl.ds(base, L)]
            yv = pts_v[pl.ds(base + 128, L)]
            ixv = (xv * INV_D + 0.5).astype(jnp.int32)
            iyv = (yv * INV_D + 0.5).astype(jnp.int32)
            m = jnp.logical_and(ixv >= lo, ixv < hi)
            r = ixv - lo
            jt4 = jnp.right_shift(iyv, 7) * CH
            jin = jnp.bitwise_and(iyv, 127)
            v2 = xv if forward else zerof
            v3 = yv if forward else zerof
            v01 = onesf if forward else zerof
            plsc.store_scatter(grid_v, [r, jt4, jin], v01, mask=m)
            plsc.store_scatter(grid_v, [r, jt4 + 1, jin], v01, mask=m)
            plsc.store_scatter(grid_v, [r, jt4 + 2, jin], v2, mask=m)
            plsc.store_scatter(grid_v, [r, jt4 + 3, jin], v3, mask=m)
            return 0
        lax.fori_loop(0, nvec, it, 0, unroll=4)

    all_has = []
    for s in range(spw):
        sbase = s * n_points * 2

        def minmax(i, carry):
            mn, mx = carry
            base = sbase + (i // 8) * 256 + (i % 8) * L
            ix = (pts_v[pl.ds(base, L)] * INV_D + 0.5).astype(jnp.int32)
            return jnp.minimum(mn, ix), jnp.maximum(mx, ix)

        big = jnp.full((L,), 2**30, jnp.int32)
        mn_v, mx_v = lax.fori_loop(0, nvec, minmax, (big, -big), unroll=4)
        mn = jnp.min(mn_v)
        mx = jnp.max(mx_v)
        all_has.append([jnp.logical_and(mx >= lo, mn < lo + h)
                        for lo, h in zip(_LOS, _HS)])

    total = spw * NPASS
    for g in range(total):
        s, p = divmod(g, NPASS)
        k = g % 2
        lo, h = _LOS[p], _HS[p]
        if g >= 2:
            ps, pp = divmod(g - 2, NPASS)
            pltpu.make_async_copy(
                grids[k].reshape(HB, 2, CH, 128).at[pl.ds(0, _HS[pp])],
                out_hbm.at[b0 + ps, pl.ds(_LOS[pp], _HS[pp])],
                sems[k]).wait()

            @pl.when(all_has[ps][pp])
            def _cleanup():
                pass_scatter(grids[k], ps, _LOS[pp],
                             _LOS[pp] + _HS[pp], False)

        @pl.when(all_has[s][p])
        def _scatter():
            pass_scatter(grids[k], s, lo, lo + h, True)

        pltpu.async_copy(
            grids[k].reshape(HB, 2, CH, 128).at[pl.ds(0, h)],
            out_hbm.at[b0 + s, pl.ds(lo, h)],
            sems[k])

    for g in (total - 2, total - 1):
        s, p = divmod(g, NPASS)
        k = g % 2
        pltpu.make_async_copy(
            grids[k].reshape(HB, 2, CH, 128).at[pl.ds(0, _HS[p])],
            out_hbm.at[b0 + s, pl.ds(_LOS[p], _HS[p])],
            sems[k]).wait()


@functools.lru_cache(maxsize=None)
def _build(batch_size, n_points):
    spw = batch_size // NW
    mesh = plsc.VectorSubcoreMesh(
        core_axis_name="c", subcore_axis_name="s",
        num_cores=NC, num_subcores=NS)
    return pl.kernel(
        functools.partial(_sc_body, n_points, spw),
        out_type=jax.ShapeDtypeStruct((batch_size, BINS0, 2, CH, 128),
                                      jnp.float32),
        mesh=mesh,
        compiler_params=pltpu.CompilerParams(needs_layout_passes=False),
        scratch_types=[
            pltpu.VMEM((spw * n_points * 2,), jnp.float32),
            pltpu.VMEM((HB, 8, 128), jnp.float32),
            pltpu.VMEM((HB, 8, 128), jnp.float32),
            pltpu.SemaphoreType.DMA,
            pltpu.SemaphoreType.DMA,
            pltpu.SemaphoreType.DMA,
        ],
    )


def kernel(batch):
    batch_size, n_points, _ = batch.shape
    flat = batch.reshape(batch_size, n_points // 128, 128, 2)
    flat = flat.transpose(0, 1, 3, 2).reshape(batch_size * n_points * 2)
    out5 = _build(batch_size, n_points)(flat)
    out = out5.transpose(0, 1, 2, 4, 3).reshape(
        batch_size, BINS0, BINS1, CH)
    return out


# speedup vs baseline: 4.4759x; 1.0023x over previous
import functools

import jax
import jax.numpy as jnp
from jax import lax
from jax.experimental import pallas as pl
from jax.experimental.pallas import tpu as pltpu
from jax.experimental.pallas import tpu_sc as plsc

BINS0, BINS1 = 256, 256
CH = 4
INV_D = 128.0
NC, NS, L = 2, 16, 16
NW = NC * NS
HB = 56
_LOS = list(range(0, BINS0, HB))
_HS = [min(HB, BINS0 - lo) for lo in _LOS]
NPASS = len(_LOS)


def _zero_fill(grid_v):
    zerof = jnp.zeros((L,), jnp.float32)

    def zrow(r, _):
        for q in range(8):
            for j in range(128 // L):
                grid_v[r, q, pl.ds(j * L, L)] = zerof
        return 0
    lax.fori_loop(0, HB, zrow, 0)


def _sc_body(n_points, spw, pts_hbm, out_hbm,
             pts_v, grid_a, grid_b, sem_a, sem_b, sem_p):
    nvec = n_points // L
    wid = lax.axis_index("s") * NC + lax.axis_index("c")
    onesf = jnp.ones((L,), jnp.float32)
    zerof = jnp.zeros((L,), jnp.float32)
    grids = [grid_a, grid_b]
    sems = [sem_a, sem_b]

    n2 = n_points * 2
    b0 = wid * spw
    cpp = pltpu.async_copy(pts_hbm.at[pl.ds(b0 * n2, spw * n2)], pts_v, sem_p)
    _zero_fill(grid_a)
    cpp.wait()

    def pass_scatter(grid_v, s, lo, hi, forward):
        sbase = s * n2

        def it(i, _):
            base = sbase + (i // 8) * 256 + (i % 8) * L
            xv = pts_v[pl.ds(base, L)]
            yv = pts_v[pl.ds(base + 128, L)]
            ixv = (xv * INV_D + 0.5).astype(jnp.int32)
            iyv = (yv * INV_D + 0.5).astype(jnp.int32)
            m = jnp.logical_and(ixv >= lo, ixv < hi)
            m = jnp.logical_and(m, jnp.right_shift(iyv, 8) == 0)
            r = ixv - lo
            jt4 = jnp.right_shift(iyv, 7) * CH
            jin = jnp.bitwise_and(iyv, 127)
            v01 = onesf if forward else zerof
            v2 = xv if forward else zerof
            v3 = yv if forward else zerof
            plsc.store_scatter(grid_v, [r, jt4, jin], v01, mask=m)
            plsc.store_scatter(grid_v, [r, jt4 + 1, jin], v01, mask=m)
            plsc.store_scatter(grid_v, [r, jt4 + 2, jin], v2, mask=m)
            plsc.store_scatter(grid_v, [r, jt4 + 3, jin], v3, mask=m)
            return 0
        lax.fori_loop(0, nvec, it, 0, unroll=4)

    all_has = []
    for s in range(spw):
        sbase = s * n2

        def minmax(i, carry):
            mn, mx = carry
            base = sbase + (i // 8) * 256 + (i % 8) * L
            ix = (pts_v[pl.ds(base, L)] * INV_D + 0.5).astype(jnp.int32)
            return jnp.minimum(mn, ix), jnp.maximum(mx, ix)

        big = jnp.full((L,), 2**30, jnp.int32)
        mn_v, mx_v = lax.fori_loop(0, nvec, minmax, (big, -big), unroll=4)
        mn = jnp.min(mn_v)
        mx = jnp.max(mx_v)
        all_has.append([jnp.logical_and(mx >= lo, mn < lo + h)
                        for lo, h in zip(_LOS, _HS)])

    total = spw * NPASS
    for g in range(total):
        s, p = divmod(g, NPASS)
        k = g % 2
        lo, h = _LOS[p], _HS[p]
        if g == 1:
            _zero_fill(grid_b)
        if g >= 2:
            ps, pp = divmod(g - 2, NPASS)
            pltpu.make_async_copy(
                grids[k].reshape(HB, 2, CH, 128).at[pl.ds(0, _HS[pp])],
                out_hbm.at[b0 + ps, pl.ds(_LOS[pp], _HS[pp])],
                sems[k]).wait()

            @pl.when(all_has[ps][pp])
            def _cleanup():
                pass_scatter(grids[k], ps, _LOS[pp],
                             _LOS[pp] + _HS[pp], False)

        @pl.when(all_has[s][p])
        def _scatter():
            pass_scatter(grids[k], s, lo, lo + h, True)

        pltpu.async_copy(
            grids[k].reshape(HB, 2, CH, 128).at[pl.ds(0, h)],
            out_hbm.at[b0 + s, pl.ds(lo, h)],
            sems[k])

    for g in (total - 2, total - 1):
        s, p = divmod(g, NPASS)
        k = g % 2
        pltpu.make_async_copy(
            grids[k].reshape(HB, 2, CH, 128).at[pl.ds(0, _HS[p])],
            out_hbm.at[b0 + s, pl.ds(_LOS[p], _HS[p])],
            sems[k]).wait()


@functools.lru_cache(maxsize=None)
def _build(batch_size, n_points):
    spw = batch_size // NW
    mesh = plsc.VectorSubcoreMesh(
        core_axis_name="c", subcore_axis_name="s",
        num_cores=NC, num_subcores=NS)
    return pl.kernel(
        functools.partial(_sc_body, n_points, spw),
        out_type=jax.ShapeDtypeStruct((batch_size, BINS0, 2, CH, 128),
                                      jnp.float32),
        mesh=mesh,
        compiler_params=pltpu.CompilerParams(needs_layout_passes=False),
        scratch_types=[
            pltpu.VMEM((spw * n_points * 2,), jnp.float32),
            pltpu.VMEM((HB, 8, 128), jnp.float32),
            pltpu.VMEM((HB, 8, 128), jnp.float32),
            pltpu.SemaphoreType.DMA,
            pltpu.SemaphoreType.DMA,
            pltpu.SemaphoreType.DMA,
        ],
    )


def kernel(batch):
    batch_size, n_points, _ = batch.shape
    flat = batch.reshape(batch_size, n_points // 128, 128, 2)
    flat = flat.transpose(0, 1, 3, 2).reshape(batch_size * n_points * 2)
    out5 = _build(batch_size, n_points)(flat)
    return out5.transpose(0, 1, 2, 4, 3).reshape(
        batch_size, BINS0, BINS1, CH)


# speedup vs baseline: 4.4848x; 1.0020x over previous
import functools

import jax
import jax.numpy as jnp
from jax import lax
from jax.experimental import pallas as pl
from jax.experimental.pallas import tpu as pltpu
from jax.experimental.pallas import tpu_sc as plsc

BINS0, BINS1 = 256, 256
CH = 4
INV_D = 128.0
NC, NS, L = 2, 16, 16
NW = NC * NS
HB = 56
_LOS = list(range(0, BINS0, HB))
_HS = [min(HB, BINS0 - lo) for lo in _LOS]
NPASS = len(_LOS)


def _zero_fill(grid_v):
    zerof = jnp.zeros((L,), jnp.float32)

    def zrow(r, _):
        for q in range(8):
            for j in range(128 // L):
                grid_v[r, q, pl.ds(j * L, L)] = zerof
        return 0
    lax.fori_loop(0, HB, zrow, 0)


def _sc_body(n_points, spw, pts_hbm, out_hbm,
             pts_v, grid_a, grid_b, sem_a, sem_b, sem_p):
    nvec = n_points // L
    wid = lax.axis_index("s") * NC + lax.axis_index("c")
    onesf = jnp.ones((L,), jnp.float32)
    zerof = jnp.zeros((L,), jnp.float32)
    grids = [grid_a, grid_b]
    sems = [sem_a, sem_b]

    n2 = n_points * 2
    b0 = wid * spw
    cpp = pltpu.async_copy(pts_hbm.at[pl.ds(b0 * n2, spw * n2)], pts_v, sem_p)
    _zero_fill(grid_a)
    cpp.wait()

    def pass_scatter(grid_v, s, lo, hi, forward):
        sbase = s * n2

        def it(i, _):
            base = sbase + (i // 8) * 256 + (i % 8) * L
            xv = pts_v[pl.ds(base, L)]
            yv = pts_v[pl.ds(base + 128, L)]
            ixv = (xv * INV_D + 0.5).astype(jnp.int32)
            iyv = (yv * INV_D + 0.5).astype(jnp.int32)
            m = jnp.logical_and(ixv >= lo, ixv < hi)
            m = jnp.logical_and(m, jnp.right_shift(iyv, 8) == 0)
            r = ixv - lo
            jt4 = jnp.right_shift(iyv, 7) * CH
            jin = jnp.bitwise_and(iyv, 127)
            v01 = onesf if forward else zerof
            v2 = xv if forward else zerof
            v3 = yv if forward else zerof
            plsc.store_scatter(grid_v, [r, jt4, jin], v01, mask=m)
            plsc.store_scatter(grid_v, [r, jt4 + 1, jin], v01, mask=m)
            plsc.store_scatter(grid_v, [r, jt4 + 2, jin], v2, mask=m)
            plsc.store_scatter(grid_v, [r, jt4 + 3, jin], v3, mask=m)
            return 0
        lax.fori_loop(0, nvec, it, 0, unroll=4)

    all_has = []
    for s in range(spw):
        sbase = s * n2

        def minmax(i, carry):
            mn, mx = carry
            base = sbase + (i // 8) * 256 + (i % 8) * L
            ix = (pts_v[pl.ds(base, L)] * INV_D + 0.5).astype(jnp.int32)
            return jnp.minimum(mn, ix), jnp.maximum(mx, ix)

        big = jnp.full((L,), 2**30, jnp.int32)
        mn_v, mx_v = lax.fori_loop(0, nvec, minmax, (big, -big), unroll=4)
        mn = jnp.min(mn_v)
        mx = jnp.max(mx_v)
        all_has.append([jnp.logical_and(mx >= lo, mn < lo + h)
                        for lo, h in zip(_LOS, _HS)])

    total = spw * NPASS
    for g in range(total):
        s, p = divmod(g, NPASS)
        k = g % 2
        lo, h = _LOS[p], _HS[p]
        if g == 1:
            _zero_fill(grid_b)
        if g >= 2:
            ps, pp = divmod(g - 2, NPASS)
            pltpu.make_async_copy(
                grids[k].reshape(HB, 2, CH, 128).at[pl.ds(0, _HS[pp])],
                out_hbm.at[b0 + ps, pl.ds(_LOS[pp], _HS[pp])],
                sems[k]).wait()

            @pl.when(all_has[ps][pp])
            def _cleanup():
                pass_scatter(grids[k], ps, _LOS[pp],
                             _LOS[pp] + _HS[pp], False)

        @pl.when(all_has[s][p])
        def _scatter():
            pass_scatter(grids[k], s, lo, lo + h, True)

        pltpu.async_copy(
            grids[k].reshape(HB, 2, CH, 128).at[pl.ds(0, h)],
            out_hbm.at[b0 + s, pl.ds(lo, h)],
            sems[k])

    for g in (total - 2, total - 1):
        s, p = divmod(g, NPASS)
        k = g % 2
        pltpu.make_async_copy(
            grids[k].reshape(HB, 2, CH, 128).at[pl.ds(0, _HS[p])],
            out_hbm.at[b0 + s, pl.ds(_LOS[p], _HS[p])],
            sems[k]).wait()


@functools.lru_cache(maxsize=None)
def _build(batch_size, n_points):
    spw = batch_size // NW
    mesh = plsc.VectorSubcoreMesh(
        core_axis_name="c", subcore_axis_name="s",
        num_cores=NC, num_subcores=NS)
    return pl.kernel(
        functools.partial(_sc_body, n_points, spw),
        out_type=jax.ShapeDtypeStruct((batch_size, BINS0, 2, CH, 128),
                                      jnp.float32),
        mesh=mesh,
        compiler_params=pltpu.CompilerParams(
            needs_layout_passes=False,
            disable_bounds_checks=True,
            disable_semaphore_checks=True),
        scratch_types=[
            pltpu.VMEM((spw * n_points * 2,), jnp.float32),
            pltpu.VMEM((HB, 8, 128), jnp.float32),
            pltpu.VMEM((HB, 8, 128), jnp.float32),
            pltpu.SemaphoreType.DMA,
            pltpu.SemaphoreType.DMA,
            pltpu.SemaphoreType.DMA,
        ],
    )


def kernel(batch):
    batch_size, n_points, _ = batch.shape
    flat = batch.reshape(batch_size, n_points // 128, 128, 2)
    flat = flat.transpose(0, 1, 3, 2).reshape(batch_size * n_points * 2)
    out5 = _build(batch_size, n_points)(flat)
    return out5.transpose(0, 1, 2, 4, 3).reshape(
        batch_size, BINS0, BINS1, CH)
